# baseline (speedup 1.0000x reference)
import functools

import jax
import jax.numpy as jnp
from jax.experimental import pallas as pl
from jax.experimental.pallas import tpu as pltpu

_LANES = 128


def _conv_kernel(x_ref, w_ref, shift_ref, o_ref, *, h, w, kh, kw):
    c_in = x_ref.shape[1]
    xb = x_ref[0].astype(jnp.bfloat16)
    lane_pad = jnp.zeros((c_in, h, _LANES - w), dtype=jnp.bfloat16)
    xw = jnp.concatenate([xb, lane_pad], axis=2)
    row_pad = jnp.zeros((c_in, kh - 1, _LANES), dtype=jnp.bfloat16)
    xp = jnp.concatenate([row_pad, xw, row_pad], axis=1)
    xflat = xp.reshape(c_in, (h + 2 * kh - 2) * _LANES)

    m = h * _LANES
    taps = []
    for i in range(kh):
        for j in range(kw):
            st = i * _LANES + j - 1 + _LANES
            taps.append(xflat[:, st:st + m])
    patches2d = jnp.concatenate(taps, axis=0)
    acc = jax.lax.dot_general(
        w_ref[...], patches2d, (((1,), (0,)), ((), ())),
        preferred_element_type=jnp.float32)
    acc = jnp.maximum(acc + shift_ref[...], 0.0)
    out3 = acc.reshape(acc.shape[0], h, _LANES)
    o_ref[0] = out3[:, :, :w]


@jax.jit
def _basic_conv2d_opt(x_nchw, weight_oihw, gamma, beta, running_mean,
                      running_var):
    eps = 1e-3
    n, c_in, h, w = x_nchw.shape
    c_out, c_in_w, kh, kw = weight_oihw.shape
    assert c_in == c_in_w and w <= _LANES - 1
    oh, ow = h, w
    k_dim = kh * kw * c_in

    scale = gamma.astype(jnp.float32) / jnp.sqrt(
        running_var.astype(jnp.float32) + eps)
    shift = beta.astype(jnp.float32) - running_mean.astype(jnp.float32) * scale
    w_t = jnp.transpose(weight_oihw, (0, 2, 3, 1)).reshape(c_out, k_dim)
    w_t = (w_t.astype(jnp.float32) * scale[:, None]).astype(jnp.bfloat16)
    shift_col = shift.reshape(c_out, 1)

    out = pl.pallas_call(
        functools.partial(_conv_kernel, h=h, w=w, kh=kh, kw=kw),
        out_shape=jax.ShapeDtypeStruct((n, c_out, oh, ow), jnp.float32),
        grid_spec=pltpu.PrefetchScalarGridSpec(
            num_scalar_prefetch=0,
            grid=(n,),
            in_specs=[
                pl.BlockSpec((1, c_in, h, w), lambda i: (i, 0, 0, 0)),
                pl.BlockSpec((c_out, k_dim), lambda i: (0, 0)),
                pl.BlockSpec((c_out, 1), lambda i: (0, 0)),
            ],
            out_specs=pl.BlockSpec((1, c_out, oh, ow), lambda i: (i, 0, 0, 0)),
        ),
        compiler_params=pltpu.CompilerParams(
            dimension_semantics=("parallel",),
            vmem_limit_bytes=64 * 1024 * 1024,
        ),
        cost_estimate=pl.CostEstimate(
            flops=2 * n * oh * ow * k_dim * c_out,
            transcendentals=0,
            bytes_accessed=n * (c_in * h * w * 4 + c_out * oh * ow * 4)
            + k_dim * c_out * 2,
        ),
    )(x_nchw, w_t, shift_col)

    return out


def kernel(x_nchw, weight_oihw, gamma, beta, running_mean, running_var):
    return _basic_conv2d_opt(x_nchw, weight_oihw, gamma, beta, running_mean,
                             running_var)


# speedup vs baseline: 3.5483x; 3.5483x over previous
import functools

import jax
import jax.numpy as jnp
from jax.experimental import pallas as pl
from jax.experimental.pallas import tpu as pltpu


def _conv_kernel(x_ref, w_ref, shift_ref, o_ref, *, kh, kw):
    h, w, c_in = x_ref.shape[1:]
    xb = x_ref[0].astype(jnp.bfloat16)
    zcol = jnp.zeros((h, 1, c_in), dtype=jnp.bfloat16)
    xw = jnp.concatenate([zcol, xb, zcol], axis=1)
    zrow = jnp.zeros((1, w + kw - 1, c_in), dtype=jnp.bfloat16)
    xp = jnp.concatenate([zrow, xw, zrow], axis=0)

    taps = [xp[i:i + h, j:j + w, :]
            for i in range(kh) for j in range(kw)]
    patches = jnp.concatenate(taps, axis=2)
    p2 = patches.reshape(h * w, kh * kw * c_in)
    acc = jnp.dot(p2, w_ref[...],
                  preferred_element_type=jnp.float32)
    acc = jnp.maximum(acc + shift_ref[...], 0.0)
    o_ref[0] = acc.reshape(h, w, acc.shape[-1])


@jax.jit
def _basic_conv2d_opt(x_nchw, weight_oihw, gamma, beta, running_mean,
                      running_var):
    eps = 1e-3
    n, c_in, h, w = x_nchw.shape
    c_out, c_in_w, kh, kw = weight_oihw.shape
    assert c_in == c_in_w
    oh, ow = h, w
    k_dim = kh * kw * c_in

    x_nhwc = jnp.transpose(x_nchw, (0, 2, 3, 1))

    scale = gamma.astype(jnp.float32) / jnp.sqrt(
        running_var.astype(jnp.float32) + eps)
    shift = beta.astype(jnp.float32) - running_mean.astype(jnp.float32) * scale
    w_mat = jnp.transpose(weight_oihw, (2, 3, 1, 0)).reshape(k_dim, c_out)
    w_mat = (w_mat.astype(jnp.float32) * scale[None, :]).astype(jnp.bfloat16)
    shift_row = shift.reshape(1, c_out)

    out_nhwc = pl.pallas_call(
        functools.partial(_conv_kernel, kh=kh, kw=kw),
        out_shape=jax.ShapeDtypeStruct((n, oh, ow, c_out), jnp.float32),
        grid_spec=pltpu.PrefetchScalarGridSpec(
            num_scalar_prefetch=0,
            grid=(n,),
            in_specs=[
                pl.BlockSpec((1, h, w, c_in), lambda i: (i, 0, 0, 0)),
                pl.BlockSpec((k_dim, c_out), lambda i: (0, 0)),
                pl.BlockSpec((1, c_out), lambda i: (0, 0)),
            ],
            out_specs=pl.BlockSpec((1, oh, ow, c_out), lambda i: (i, 0, 0, 0)),
        ),
        compiler_params=pltpu.CompilerParams(
            dimension_semantics=("parallel",),
            vmem_limit_bytes=64 * 1024 * 1024,
        ),
        cost_estimate=pl.CostEstimate(
            flops=2 * n * oh * ow * k_dim * c_out,
            transcendentals=0,
            bytes_accessed=n * (h * w * c_in * 4 + oh * ow * c_out * 4)
            + k_dim * c_out * 2,
        ),
    )(x_nhwc, w_mat, shift_row)

    return jnp.transpose(out_nhwc, (0, 3, 1, 2))


def kernel(x_nchw, weight_oihw, gamma, beta, running_mean, running_var):
    return _basic_conv2d_opt(x_nchw, weight_oihw, gamma, beta, running_mean,
                             running_var)


# speedup vs baseline: 4.1745x; 1.1765x over previous
import functools

import jax
import jax.numpy as jnp
from jax.experimental import pallas as pl
from jax.experimental.pallas import tpu as pltpu


def _conv_kernel(x_ref, w_ref, shift_ref, o_ref, *, kh, kw):
    b, h, w, c_in = x_ref.shape
    for bi in range(b):
        xb = x_ref[bi].astype(jnp.bfloat16)
        zcol = jnp.zeros((h, 1, c_in), dtype=jnp.bfloat16)
        xw = jnp.concatenate([zcol, xb, zcol], axis=1)
        zrow = jnp.zeros((1, w + kw - 1, c_in), dtype=jnp.bfloat16)
        xp = jnp.concatenate([zrow, xw, zrow], axis=0)
        taps = [xp[i:i + h, j:j + w, :]
                for i in range(kh) for j in range(kw)]
        patches = jnp.concatenate(taps, axis=2)
        p2 = patches.reshape(h * w, kh * kw * c_in)
        acc = jnp.dot(p2, w_ref[...],
                      preferred_element_type=jnp.float32)
        acc = jnp.maximum(acc + shift_ref[...], 0.0)
        o_ref[bi] = acc.reshape(h, w, acc.shape[-1])


@jax.jit
def _basic_conv2d_opt(x_nchw, weight_oihw, gamma, beta, running_mean,
                      running_var):
    eps = 1e-3
    n, c_in, h, w = x_nchw.shape
    c_out, c_in_w, kh, kw = weight_oihw.shape
    assert c_in == c_in_w
    oh, ow = h, w
    k_dim = kh * kw * c_in
    blk = 2 if n % 2 == 0 else 1

    x_nhwc = jnp.transpose(x_nchw, (0, 2, 3, 1))

    scale = gamma.astype(jnp.float32) / jnp.sqrt(
        running_var.astype(jnp.float32) + eps)
    shift = beta.astype(jnp.float32) - running_mean.astype(jnp.float32) * scale
    w_mat = jnp.transpose(weight_oihw, (2, 3, 1, 0)).reshape(k_dim, c_out)
    w_mat = (w_mat.astype(jnp.float32) * scale[None, :]).astype(jnp.bfloat16)
    shift_row = shift.reshape(1, c_out)

    out_nhwc = pl.pallas_call(
        functools.partial(_conv_kernel, kh=kh, kw=kw),
        out_shape=jax.ShapeDtypeStruct((n, oh, ow, c_out), jnp.float32),
        grid_spec=pltpu.PrefetchScalarGridSpec(
            num_scalar_prefetch=0,
            grid=(n // blk,),
            in_specs=[
                pl.BlockSpec((blk, h, w, c_in), lambda i: (i, 0, 0, 0)),
                pl.BlockSpec((k_dim, c_out), lambda i: (0, 0)),
                pl.BlockSpec((1, c_out), lambda i: (0, 0)),
            ],
            out_specs=pl.BlockSpec((blk, oh, ow, c_out),
                                   lambda i: (i, 0, 0, 0)),
        ),
        compiler_params=pltpu.CompilerParams(
            dimension_semantics=("parallel",),
            vmem_limit_bytes=64 * 1024 * 1024,
        ),
        cost_estimate=pl.CostEstimate(
            flops=2 * n * oh * ow * k_dim * c_out,
            transcendentals=0,
            bytes_accessed=n * (h * w * c_in * 4 + oh * ow * c_out * 4)
            + k_dim * c_out * 2,
        ),
    )(x_nhwc, w_mat, shift_row)

    return jnp.transpose(out_nhwc, (0, 3, 1, 2))


def kernel(x_nchw, weight_oihw, gamma, beta, running_mean, running_var):
    return _basic_conv2d_opt(x_nchw, weight_oihw, gamma, beta, running_mean,
                             running_var)


# speedup vs baseline: 4.4618x; 1.0688x over previous
import functools

import jax
import jax.numpy as jnp
from jax.experimental import pallas as pl
from jax.experimental.pallas import tpu as pltpu


def _conv_kernel(x_ref, w_ref, shift_ref, o_ref, *, kh, kw):
    b, h, w, c_in = x_ref.shape
    for bi in range(b):
        xb = x_ref[bi].astype(jnp.bfloat16)
        zcol = jnp.zeros((h, 1, c_in), dtype=jnp.bfloat16)
        xw = jnp.concatenate([zcol, xb, zcol], axis=1)
        zrow = jnp.zeros((1, w + kw - 1, c_in), dtype=jnp.bfloat16)
        xp = jnp.concatenate([zrow, xw, zrow], axis=0)
        taps = [xp[i:i + h, j:j + w, :]
                for i in range(kh) for j in range(kw)]
        patches = jnp.concatenate(taps, axis=2)
        p2 = patches.reshape(h * w, kh * kw * c_in)
        acc = jnp.dot(p2, w_ref[...],
                      preferred_element_type=jnp.float32)
        acc = jnp.maximum(acc + shift_ref[...], 0.0)
        o_ref[bi] = acc.reshape(h, w, acc.shape[-1])


@jax.jit
def _basic_conv2d_opt(x_nchw, weight_oihw, gamma, beta, running_mean,
                      running_var):
    eps = 1e-3
    n, c_in, h, w = x_nchw.shape
    c_out, c_in_w, kh, kw = weight_oihw.shape
    assert c_in == c_in_w
    oh, ow = h, w
    k_dim = kh * kw * c_in
    blk = 4 if n % 4 == 0 else (2 if n % 2 == 0 else 1)

    x_nhwc = jnp.transpose(x_nchw, (0, 2, 3, 1))

    scale = gamma.astype(jnp.float32) / jnp.sqrt(
        running_var.astype(jnp.float32) + eps)
    shift = beta.astype(jnp.float32) - running_mean.astype(jnp.float32) * scale
    w_mat = jnp.transpose(weight_oihw, (2, 3, 1, 0)).reshape(k_dim, c_out)
    w_mat = (w_mat.astype(jnp.float32) * scale[None, :]).astype(jnp.bfloat16)
    shift_row = shift.reshape(1, c_out)

    out_nhwc = pl.pallas_call(
        functools.partial(_conv_kernel, kh=kh, kw=kw),
        out_shape=jax.ShapeDtypeStruct((n, oh, ow, c_out), jnp.float32),
        grid_spec=pltpu.PrefetchScalarGridSpec(
            num_scalar_prefetch=0,
            grid=(n // blk,),
            in_specs=[
                pl.BlockSpec((blk, h, w, c_in), lambda i: (i, 0, 0, 0)),
                pl.BlockSpec((k_dim, c_out), lambda i: (0, 0)),
                pl.BlockSpec((1, c_out), lambda i: (0, 0)),
            ],
            out_specs=pl.BlockSpec((blk, oh, ow, c_out),
                                   lambda i: (i, 0, 0, 0)),
        ),
        compiler_params=pltpu.CompilerParams(
            dimension_semantics=("parallel",),
            vmem_limit_bytes=64 * 1024 * 1024,
        ),
        cost_estimate=pl.CostEstimate(
            flops=2 * n * oh * ow * k_dim * c_out,
            transcendentals=0,
            bytes_accessed=n * (h * w * c_in * 4 + oh * ow * c_out * 4)
            + k_dim * c_out * 2,
        ),
    )(x_nhwc, w_mat, shift_row)

    return jnp.transpose(out_nhwc, (0, 3, 1, 2))


def kernel(x_nchw, weight_oihw, gamma, beta, running_mean, running_var):
    return _basic_conv2d_opt(x_nchw, weight_oihw, gamma, beta, running_mean,
                             running_var)
